# Initial kernel scaffold; baseline (speedup 1.0000x reference)
import functools

import jax
import jax.numpy as jnp
from jax import lax
from jax.experimental import pallas as pl
from jax.experimental.pallas import tpu as pltpu
from jax.experimental.pallas import tpu_sc as plsc

N = 10000
D = 128
EL = 320000
EG = 160000
NC = 2
NS = 16
NPAD = 10240
K = 256
NCH_L = EL // K
NCH_G = EG // K


def _rsqrt16(x):
    i = plsc.bitcast(x, jnp.int32)
    y = plsc.bitcast(jnp.int32(0x5F3759DF) - lax.shift_right_logical(i, 1),
                     jnp.float32)
    for _ in range(3):
        y = y * (1.5 - 0.5 * x * y * y)
    return y


def _sc_body(src_l, dst_l, ew_l, src_g, dst_g, ew_g, h_l, h_g, parts,
             acc_s, dgl_s, dgg_s, dinv_l_v, dinv_g_v, zv_v,
             si_v, di_v, ew_v, nrm_v, rows_v, sem):
    cid = lax.axis_index("c")
    sid = lax.axis_index("s")

    z16 = jnp.zeros((16,), jnp.float32)
    for i in range(40):
        zv_v[pl.ds(16 * i, 16)] = z16
    pltpu.sync_copy(zv_v, dgl_s.at[pl.ds(640 * sid, 640)])
    pltpu.sync_copy(zv_v, dgg_s.at[pl.ds(640 * sid, 640)])
    plsc.subcore_barrier()

    def deg_loop(dst2, ew2, deg_s, nch):
        trips = (nch - sid + NS - 1) // NS

        def body(k, _):
            c = sid + NS * k
            pltpu.sync_copy(dst2.at[pl.ds(2 * c, 2)], di_v)
            pltpu.sync_copy(ew2.at[pl.ds(2 * c, 2)], ew_v)
            for j in range(2):
                pltpu.sync_copy(ew_v.at[j], deg_s.at[di_v.at[j]], add=True)
            return 0

        lax.fori_loop(0, trips, body, 0)

    deg_loop(dst_l, ew_l, dgl_s, NCH_L)
    deg_loop(dst_g, ew_g, dgg_s, NCH_G)
    plsc.subcore_barrier()

    pltpu.sync_copy(dgl_s, dinv_l_v)
    pltpu.sync_copy(dgg_s, dinv_g_v)

    def newton(dv):
        def body(i, _):
            sl = pl.ds(16 * i, 16)
            dv[sl] = _rsqrt16(dv[sl] + 1.0)
            return 0

        lax.fori_loop(0, NPAD // 16, body, 0)

    newton(dinv_l_v)
    newton(dinv_g_v)

    def selfloop(h_hbm, dinv_v):
        for t in range(5):
            base = 625 * sid + 125 * t
            pltpu.sync_copy(h_hbm.at[pl.ds(base, 125)], rows_v.at[pl.ds(0, 125)])

            def srow(r, _):
                dvec = plsc.load_gather(
                    dinv_v, [jnp.full((16,), base + r, jnp.int32)])
                sq = dvec * dvec
                for j in range(8):
                    sl = pl.ds(16 * j, 16)
                    rows_v[r, sl] = rows_v[r, sl] * sq
                return 0

            lax.fori_loop(0, 125, srow, 0)
            pltpu.sync_copy(rows_v.at[pl.ds(0, 125)], acc_s.at[pl.ds(base, 125)])

    pl.when(cid == 0)(lambda: selfloop(h_l, dinv_l_v))
    pl.when(cid == 1)(lambda: selfloop(h_g, dinv_g_v))
    plsc.subcore_barrier()

    def msg_loop(src2, dst2, ew2, h_hbm, dinv_v, start, count_sc):
        trips = (count_sc - sid + NS - 1) // NS

        def body(k, _):
            c = start + sid + NS * k
            pltpu.sync_copy(src2.at[pl.ds(2 * c, 2)], si_v)
            pltpu.sync_copy(dst2.at[pl.ds(2 * c, 2)], di_v)
            pltpu.sync_copy(ew2.at[pl.ds(2 * c, 2)], ew_v)
            cp0 = pltpu.async_copy(h_hbm.at[si_v.at[0]],
                                   rows_v.at[pl.ds(0, 128)], sem)
            cp1 = pltpu.async_copy(h_hbm.at[si_v.at[1]],
                                   rows_v.at[pl.ds(128, 128)], sem)
            for i in range(16):
                r, o = divmod(i, 8)
                sl = pl.ds(16 * o, 16)
                s16 = si_v[r, sl]
                d16 = di_v[r, sl]
                w16 = ew_v[r, sl]
                dl = plsc.load_gather(dinv_v, [s16])
                dr = plsc.load_gather(dinv_v, [d16])
                nrm_v[pl.ds(16 * i, 16)] = dl * w16 * dr
            cp0.wait()
            cp1.wait()

            def srow(e, _):
                nb = plsc.load_gather(nrm_v, [jnp.full((16,), e, jnp.int32)])
                for j in range(8):
                    sl = pl.ds(16 * j, 16)
                    rows_v[e, sl] = rows_v[e, sl] * nb
                return 0

            lax.fori_loop(0, K, srow, 0)
            pltpu.sync_copy(rows_v.at[pl.ds(0, 128)],
                            acc_s.at[di_v.at[0]], add=True)
            pltpu.sync_copy(rows_v.at[pl.ds(128, 128)],
                            acc_s.at[di_v.at[1]], add=True)
            return 0

        lax.fori_loop(0, trips, body, 0)

    msg_loop(src_l, dst_l, ew_l, h_l, dinv_l_v, cid * (NCH_L // 2), NCH_L // 2)
    msg_loop(src_g, dst_g, ew_g, h_g, dinv_g_v, cid * (NCH_G // 2),
             NCH_G // 2 + cid)
    plsc.subcore_barrier()

    pltpu.sync_copy(acc_s.at[pl.ds(625 * sid, 625)],
                    parts.at[cid, pl.ds(625 * sid, 625)])


def _sc_conv(src_l, dst_l, ew_l, src_g, dst_g, ew_g, h_l, h_g):
    mesh = plsc.VectorSubcoreMesh(core_axis_name="c", subcore_axis_name="s")
    return pl.kernel(
        _sc_body,
        out_type=jax.ShapeDtypeStruct((NC, N, D), jnp.float32),
        mesh=mesh,
        scratch_types=[
            pltpu.VMEM_SHARED((N, D), jnp.float32),
            pltpu.VMEM_SHARED((NPAD,), jnp.float32),
            pltpu.VMEM_SHARED((NPAD,), jnp.float32),
            pltpu.VMEM((NPAD,), jnp.float32),
            pltpu.VMEM((NPAD,), jnp.float32),
            pltpu.VMEM((640,), jnp.float32),
            pltpu.VMEM((2, 128), jnp.int32),
            pltpu.VMEM((2, 128), jnp.int32),
            pltpu.VMEM((2, 128), jnp.float32),
            pltpu.VMEM((K,), jnp.float32),
            pltpu.VMEM((K, D), jnp.float32),
            pltpu.SemaphoreType.DMA,
        ],
    )(src_l, dst_l, ew_l, src_g, dst_g, ew_g, h_l, h_g)


def _mm_body(a_ref, l_ref, wl_ref, wg_ref, hl_ref, hg_ref):
    xa = a_ref[...] + l_ref[...]
    hl_ref[...] = jnp.dot(xa, wl_ref[...], preferred_element_type=jnp.float32)
    hg_ref[...] = jnp.dot(xa, wg_ref[...], preferred_element_type=jnp.float32)


def _tc_matmul(activity, learning, W_l2, W_g2):
    BM = 1000
    grid = (N // BM,)
    return pl.pallas_call(
        _mm_body,
        grid=grid,
        in_specs=[
            pl.BlockSpec((BM, D), lambda i: (i, 0)),
            pl.BlockSpec((BM, D), lambda i: (i, 0)),
            pl.BlockSpec((D, D), lambda i: (0, 0)),
            pl.BlockSpec((D, D), lambda i: (0, 0)),
        ],
        out_specs=[
            pl.BlockSpec((BM, D), lambda i: (i, 0)),
            pl.BlockSpec((BM, D), lambda i: (i, 0)),
        ],
        out_shape=[
            jax.ShapeDtypeStruct((N, D), jnp.float32),
            jax.ShapeDtypeStruct((N, D), jnp.float32),
        ],
    )(activity, learning, W_l2, W_g2)


def _fin_body(p0_ref, p1_ref, a_ref, l_ref, bl_ref, bg_ref, o_ref):
    xlg = p0_ref[...] + p1_ref[...] + (bl_ref[...] + bg_ref[...])
    wei = jax.nn.sigmoid(xlg)
    o_ref[...] = 2.0 * a_ref[...] * wei + 2.0 * l_ref[...] * (1.0 - wei)


def _tc_final(p0, p1, activity, learning, b_l2, b_g2):
    BM = 1000
    grid = (N // BM,)
    row = lambda i: (i, 0)
    fix = lambda i: (0, 0)
    return pl.pallas_call(
        _fin_body,
        grid=grid,
        in_specs=[
            pl.BlockSpec((BM, D), row),
            pl.BlockSpec((BM, D), row),
            pl.BlockSpec((BM, D), row),
            pl.BlockSpec((BM, D), row),
            pl.BlockSpec((1, D), fix),
            pl.BlockSpec((1, D), fix),
        ],
        out_specs=pl.BlockSpec((BM, D), row),
        out_shape=jax.ShapeDtypeStruct((N, D), jnp.float32),
    )(p0, p1, activity, learning, b_l2, b_g2)


def kernel(activity, learning, edge_index, edge_weight,
           global_edge_index, global_edge_weight,
           W_l1, b_l1, W_l2, b_l2, W_g1, b_g1, W_g2, b_g2):
    h_l, h_g = _tc_matmul(activity, learning, W_l2, W_g2)

    src_l = edge_index[0].reshape(2 * NCH_L, 128)
    dst_l = edge_index[1].reshape(2 * NCH_L, 128)
    ew_l = edge_weight.reshape(2 * NCH_L, 128)
    src_g = global_edge_index[0].reshape(2 * NCH_G, 128)
    dst_g = global_edge_index[1].reshape(2 * NCH_G, 128)
    ew_g = global_edge_weight.reshape(2 * NCH_G, 128)

    parts = _sc_conv(src_l, dst_l, ew_l, src_g, dst_g, ew_g, h_l, h_g)

    return _tc_final(parts[0], parts[1], activity, learning,
                     b_l2.reshape(1, D), b_g2.reshape(1, D))


# speedup vs baseline: 10.7812x; 10.7812x over previous
import functools

import jax
import jax.numpy as jnp
from jax import lax
from jax.experimental import pallas as pl
from jax.experimental.pallas import tpu as pltpu
from jax.experimental.pallas import tpu_sc as plsc

N = 10000
D = 128
EL = 320000
EG = 160000
NC = 2
NS = 16
NPAD = 10240
K = 128
NCH_L = EL // K
NCH_G = EG // K


def _rsqrt16(x):
    i = plsc.bitcast(x, jnp.int32)
    y = plsc.bitcast(jnp.int32(0x5F3759DF) - lax.shift_right_logical(i, 1),
                     jnp.float32)
    for _ in range(3):
        y = y * (1.5 - 0.5 * x * y * y)
    return y


def _sc_body(src_l, dst_l, ew_l, src_g, dst_g, ew_g, h_l, h_g, parts,
             acc_s, dgl_s, dgg_s, dinv_l_v, dinv_g_v, zv_v,
             si_v, di_v, ew_v, nrm_v, rows_v, sem):
    cid = lax.axis_index("c")
    sid = lax.axis_index("s")

    z16 = jnp.zeros((16,), jnp.float32)
    for i in range(40):
        zv_v[pl.ds(16 * i, 16)] = z16
    pltpu.sync_copy(zv_v, dgl_s.at[pl.ds(640 * sid, 640)])
    pltpu.sync_copy(zv_v, dgg_s.at[pl.ds(640 * sid, 640)])
    plsc.subcore_barrier()

    def deg_loop(dst2, ew2, deg_s, nch):
        trips = (nch - sid + NS - 1) // NS

        def body(k, _):
            c = sid + NS * k
            pltpu.sync_copy(dst2.at[pl.ds(c, 1)], di_v)
            pltpu.sync_copy(ew2.at[pl.ds(c, 1)], ew_v)
            pltpu.sync_copy(ew_v.at[0], deg_s.at[di_v.at[0]], add=True)
            return 0

        lax.fori_loop(0, trips, body, 0)

    deg_loop(dst_l, ew_l, dgl_s, NCH_L)
    deg_loop(dst_g, ew_g, dgg_s, NCH_G)
    plsc.subcore_barrier()

    pltpu.sync_copy(dgl_s, dinv_l_v)
    pltpu.sync_copy(dgg_s, dinv_g_v)

    def newton(dv):
        def body(i, _):
            sl = pl.ds(16 * i, 16)
            dv[sl] = _rsqrt16(dv[sl] + 1.0)
            return 0

        lax.fori_loop(0, NPAD // 16, body, 0)

    newton(dinv_l_v)
    newton(dinv_g_v)

    def selfloop(h_hbm, dinv_v):
        for t in range(5):
            base = 625 * sid + 125 * t
            pltpu.sync_copy(h_hbm.at[pl.ds(base, 125)], rows_v.at[pl.ds(0, 125)])

            def srow(r, _):
                dvec = plsc.load_gather(
                    dinv_v, [jnp.full((16,), base + r, jnp.int32)])
                sq = dvec * dvec
                for j in range(8):
                    sl = pl.ds(16 * j, 16)
                    rows_v[r, sl] = rows_v[r, sl] * sq
                return 0

            lax.fori_loop(0, 125, srow, 0)
            pltpu.sync_copy(rows_v.at[pl.ds(0, 125)], acc_s.at[pl.ds(base, 125)])

    pl.when(cid == 0)(lambda: selfloop(h_l, dinv_l_v))
    pl.when(cid == 1)(lambda: selfloop(h_g, dinv_g_v))
    plsc.subcore_barrier()

    def msg_loop(src2, dst2, ew2, h_hbm, dinv_v, start, count_sc):
        trips = (count_sc - sid + NS - 1) // NS

        def body(k, _):
            c = start + sid + NS * k
            pltpu.sync_copy(src2.at[pl.ds(c, 1)], si_v)
            pltpu.sync_copy(dst2.at[pl.ds(c, 1)], di_v)
            pltpu.sync_copy(ew2.at[pl.ds(c, 1)], ew_v)
            cp0 = pltpu.async_copy(h_hbm.at[si_v.at[0]], rows_v, sem)
            for i in range(8):
                sl = pl.ds(16 * i, 16)
                s16 = si_v[0, sl]
                d16 = di_v[0, sl]
                w16 = ew_v[0, sl]
                dl = plsc.load_gather(dinv_v, [s16])
                dr = plsc.load_gather(dinv_v, [d16])
                nrm_v[pl.ds(16 * i, 16)] = dl * w16 * dr
            cp0.wait()

            def srow(e, _):
                nb = plsc.load_gather(nrm_v, [jnp.full((16,), e, jnp.int32)])
                for j in range(8):
                    sl = pl.ds(16 * j, 16)
                    rows_v[e, sl] = rows_v[e, sl] * nb
                return 0

            lax.fori_loop(0, K, srow, 0)
            pltpu.sync_copy(rows_v, acc_s.at[di_v.at[0]], add=True)
            return 0

        lax.fori_loop(0, trips, body, 0)

    msg_loop(src_l, dst_l, ew_l, h_l, dinv_l_v, cid * (NCH_L // 2), NCH_L // 2)
    msg_loop(src_g, dst_g, ew_g, h_g, dinv_g_v, cid * (NCH_G // 2), NCH_G // 2)
    plsc.subcore_barrier()

    pltpu.sync_copy(acc_s.at[pl.ds(625 * sid, 625)],
                    parts.at[cid, pl.ds(625 * sid, 625)])


def _sc_conv(src_l, dst_l, ew_l, src_g, dst_g, ew_g, h_l, h_g):
    mesh = plsc.VectorSubcoreMesh(core_axis_name="c", subcore_axis_name="s")
    return pl.kernel(
        _sc_body,
        out_type=jax.ShapeDtypeStruct((NC, N, D), jnp.float32),
        mesh=mesh,
        compiler_params=pltpu.CompilerParams(use_tc_tiling_on_sc=False,
                                             needs_layout_passes=False),
        scratch_types=[
            pltpu.VMEM_SHARED((N, D), jnp.float32),
            pltpu.VMEM_SHARED((NPAD,), jnp.float32),
            pltpu.VMEM_SHARED((NPAD,), jnp.float32),
            pltpu.VMEM((NPAD,), jnp.float32),
            pltpu.VMEM((NPAD,), jnp.float32),
            pltpu.VMEM((640,), jnp.float32),
            pltpu.VMEM((1, 128), jnp.int32),
            pltpu.VMEM((1, 128), jnp.int32),
            pltpu.VMEM((1, 128), jnp.float32),
            pltpu.VMEM((K,), jnp.float32),
            pltpu.VMEM((K, D), jnp.float32),
            pltpu.SemaphoreType.DMA,
        ],
    )(src_l, dst_l, ew_l, src_g, dst_g, ew_g, h_l, h_g)


def _mm_body(a_ref, l_ref, wl_ref, wg_ref, hl_ref, hg_ref):
    xa = a_ref[...] + l_ref[...]
    hl_ref[...] = jnp.dot(xa, wl_ref[...], preferred_element_type=jnp.float32)
    hg_ref[...] = jnp.dot(xa, wg_ref[...], preferred_element_type=jnp.float32)


def _tc_matmul(activity, learning, W_l2, W_g2):
    BM = 1000
    grid = (N // BM,)
    return pl.pallas_call(
        _mm_body,
        grid=grid,
        in_specs=[
            pl.BlockSpec((BM, D), lambda i: (i, 0)),
            pl.BlockSpec((BM, D), lambda i: (i, 0)),
            pl.BlockSpec((D, D), lambda i: (0, 0)),
            pl.BlockSpec((D, D), lambda i: (0, 0)),
        ],
        out_specs=[
            pl.BlockSpec((BM, D), lambda i: (i, 0)),
            pl.BlockSpec((BM, D), lambda i: (i, 0)),
        ],
        out_shape=[
            jax.ShapeDtypeStruct((N, D), jnp.float32),
            jax.ShapeDtypeStruct((N, D), jnp.float32),
        ],
    )(activity, learning, W_l2, W_g2)


def _fin_body(p0_ref, p1_ref, a_ref, l_ref, bl_ref, bg_ref, o_ref):
    xlg = p0_ref[...] + p1_ref[...] + (bl_ref[...] + bg_ref[...])
    wei = jax.nn.sigmoid(xlg)
    o_ref[...] = 2.0 * a_ref[...] * wei + 2.0 * l_ref[...] * (1.0 - wei)


def _tc_final(p0, p1, activity, learning, b_l2, b_g2):
    BM = 1000
    grid = (N // BM,)
    row = lambda i: (i, 0)
    fix = lambda i: (0, 0)
    return pl.pallas_call(
        _fin_body,
        grid=grid,
        in_specs=[
            pl.BlockSpec((BM, D), row),
            pl.BlockSpec((BM, D), row),
            pl.BlockSpec((BM, D), row),
            pl.BlockSpec((BM, D), row),
            pl.BlockSpec((1, D), fix),
            pl.BlockSpec((1, D), fix),
        ],
        out_specs=pl.BlockSpec((BM, D), row),
        out_shape=jax.ShapeDtypeStruct((N, D), jnp.float32),
    )(p0, p1, activity, learning, b_l2, b_g2)


def kernel(activity, learning, edge_index, edge_weight,
           global_edge_index, global_edge_weight,
           W_l1, b_l1, W_l2, b_l2, W_g1, b_g1, W_g2, b_g2):
    h_l, h_g = _tc_matmul(activity, learning, W_l2, W_g2)

    src_l = edge_index[0].reshape(NCH_L, 128)
    dst_l = edge_index[1].reshape(NCH_L, 128)
    ew_l = edge_weight.reshape(NCH_L, 128)
    src_g = global_edge_index[0].reshape(NCH_G, 128)
    dst_g = global_edge_index[1].reshape(NCH_G, 128)
    ew_g = global_edge_weight.reshape(NCH_G, 128)

    parts = _sc_conv(src_l, dst_l, ew_l, src_g, dst_g, ew_g, h_l, h_g)

    return _tc_final(parts[0], parts[1], activity, learning,
                     b_l2.reshape(1, D), b_g2.reshape(1, D))


# speedup vs baseline: 25.0978x; 2.3279x over previous
import jax
import jax.numpy as jnp
from jax import lax
from jax.experimental import pallas as pl
from jax.experimental.pallas import tpu as pltpu
from jax.experimental.pallas import tpu_sc as plsc

N = 10000
D = 128
EL = 320000
EG = 160000
NC = 2
NS = 16
NPAD = 10240
K = 128
NCH_L = EL // K
NCH_G = EG // K
NCH = NCH_L + NCH_G
CPS = NCH // NC
DG = NCH // 2


def _rsqrt16(x):
    i = plsc.bitcast(x, jnp.int32)
    y = plsc.bitcast(jnp.int32(0x5F3759DF) - lax.shift_right_logical(i, 1),
                     jnp.float32)
    for _ in range(3):
        y = y * (1.5 - 0.5 * x * y * y)
    return y


def _bcast(v16, lane):
    return jnp.take_along_axis(v16, jnp.full((16,), lane, jnp.int32), axis=0,
                               mode="promise_in_bounds")


def _sc_body(src_all, dst_all, ew_all, h_all, parts, dinv_hbm, *s):
    (acc_s, dgl_s, dgg_s, zv, nbuf, dvrow, ii,
     dd0, dd1, ee0, ee1,
     si0, di0, ew0, hix0, six0, dix0, dvs0, dvd0, nrm0, rows0,
     si1, di1, ew1, hix1, six1, dix1, dvs1, dvd1, nrm1, rows1,
     isem0, isem1, gsem0, gsem1, ssem0, ssem1, zsem) = s
    dd = (dd0, dd1)
    ee = (ee0, ee1)
    si = (si0, si1)
    di = (di0, di1)
    ew = (ew0, ew1)
    hix = (hix0, hix1)
    six = (six0, six1)
    dix = (dix0, dix1)
    dvs = (dvs0, dvs1)
    dvd = (dvd0, dvd1)
    nrm = (nrm0, nrm1)
    rows = (rows0, rows1)
    isem = (isem0, isem1)
    gsem = (gsem0, gsem1)
    ssem = (ssem0, ssem1)

    cid = lax.axis_index("c")
    sid = lax.axis_index("s")

    z16 = jnp.zeros((16,), jnp.float32)
    for i in range(40):
        zv[pl.ds(16 * i, 16)] = z16
    pltpu.sync_copy(zv, dgl_s.at[pl.ds(640 * sid, 640)])
    pltpu.sync_copy(zv, dgg_s.at[pl.ds(640 * sid, 640)])

    def zrow(r, _):
        for j in range(8):
            rows0[r, pl.ds(16 * j, 16)] = z16
        return 0

    lax.fori_loop(0, 128, zrow, 0)
    zcps = [pltpu.async_copy(rows0, acc_s.at[pl.ds(640 * sid + 128 * t, 128)],
                             zsem) for t in range(5)]
    plsc.subcore_barrier()

    dtrips = (DG - sid + NS - 1) // NS

    def dfire(k, b):
        g = sid + NS * k
        pltpu.async_copy(dst_all.at[pl.ds(2 * g, 2)], dd[b], isem[b])
        pltpu.async_copy(ew_all.at[pl.ds(2 * g, 2)], ee[b], isem[b])

    def dconsume(k, b):
        g = sid + NS * k
        pltpu.make_async_copy(dst_all.at[pl.ds(2 * g, 2)], dd[b],
                              isem[b]).wait()
        pltpu.make_async_copy(ew_all.at[pl.ds(2 * g, 2)], ee[b],
                              isem[b]).wait()

        def to(tgt):
            for j in range(2):
                pltpu.sync_copy(ee[b].at[j], tgt.at[dd[b].at[j]], add=True)

        pl.when(g < NCH_L // 2)(lambda: to(dgl_s))
        pl.when(g >= NCH_L // 2)(lambda: to(dgg_s))

    dfire(0, 0)

    def dpair(t, _):
        k0 = 2 * t
        k1 = k0 + 1
        pl.when(k1 < dtrips)(lambda: dfire(k1, 1))
        dconsume(k0, 0)
        pl.when(k0 + 2 < dtrips)(lambda: dfire(k0 + 2, 0))
        pl.when(k1 < dtrips)(lambda: dconsume(k1, 1))
        return 0

    lax.fori_loop(0, (dtrips + 1) // 2, dpair, 0)

    for cp in zcps:
        cp.wait()
    plsc.subcore_barrier()

    pl.when(sid < 8)(
        lambda: pltpu.sync_copy(dgl_s.at[pl.ds(1280 * sid, 1280)], nbuf))
    pl.when(sid >= 8)(
        lambda: pltpu.sync_copy(dgg_s.at[pl.ds(1280 * sid - NPAD, 1280)],
                                nbuf))

    def nt(i, _):
        sl = pl.ds(16 * i, 16)
        nbuf[sl] = _rsqrt16(nbuf[sl] + 1.0)
        return 0

    lax.fori_loop(0, 80, nt, 0)
    pltpu.sync_copy(nbuf,
                    dinv_hbm.at[pl.ds(cid * 2 * NPAD + 1280 * sid, 1280)])
    plsc.subcore_barrier()

    offh = cid * N
    offd = cid * (2 * NPAD) + cid * NPAD
    iota16 = lax.broadcasted_iota(jnp.int32, (16,), 0)
    for t in range(5):
        base = 640 * sid + 128 * t
        pltpu.sync_copy(dinv_hbm.at[pl.ds(offd + base, 128)], dvrow)
        for i in range(8):
            sl = pl.ds(16 * i, 16)
            row16 = iota16 + (base + 16 * i)
            ii[0, sl] = row16
            hix0[0, sl] = jnp.minimum(row16, N - 1) + offh
        pltpu.async_copy(h_all.at[hix0.at[0]], rows0, gsem0).wait()

        def srow2(q, _):
            d16 = dvrow[pl.ds(16 * q, 16)]
            sq = d16 * d16
            for e0 in range(16):
                nb = _bcast(sq, e0)
                r = 16 * q + e0
                for j in range(8):
                    sl2 = pl.ds(16 * j, 16)
                    rows0[r, sl2] = rows0[r, sl2] * nb
            return 0

        lax.fori_loop(0, 8, srow2, 0)
        pltpu.sync_copy(rows0, acc_s.at[ii.at[0]], add=True)

    cbase = cid * CPS + sid
    trips = (CPS - sid + NS - 1) // NS

    def fire(k, b):
        pl.when(k >= 2)(lambda: pltpu.make_async_copy(
            rows[b], acc_s.at[di[b].at[0]], ssem[b]).wait())
        c = cbase + NS * k
        a0 = pltpu.async_copy(src_all.at[pl.ds(c, 1)], si[b], isem[b])
        a1 = pltpu.async_copy(dst_all.at[pl.ds(c, 1)], di[b], isem[b])
        a2 = pltpu.async_copy(ew_all.at[pl.ds(c, 1)], ew[b], isem[b])
        a0.wait()
        a1.wait()
        a2.wait()
        g = jnp.where(c >= NCH_L, jnp.int32(1), jnp.int32(0))
        oh = g * N
        od = g * NPAD + cid * (2 * NPAD)
        for i in range(8):
            sl = pl.ds(16 * i, 16)
            s16 = si[b][0, sl]
            d16 = di[b][0, sl]
            hix[b][0, sl] = s16 + oh
            six[b][0, sl] = s16 + od
            dix[b][0, sl] = d16 + od
        pltpu.async_copy(h_all.at[hix[b].at[0]], rows[b], gsem[b])
        pltpu.async_copy(dinv_hbm.at[six[b].at[0]], dvs[b], gsem[b])
        pltpu.async_copy(dinv_hbm.at[dix[b].at[0]], dvd[b], gsem[b])

    def consume(b):
        pltpu.make_async_copy(h_all.at[hix[b].at[0]], rows[b],
                              gsem[b]).wait()
        pltpu.make_async_copy(dinv_hbm.at[six[b].at[0]], dvs[b],
                              gsem[b]).wait()
        pltpu.make_async_copy(dinv_hbm.at[dix[b].at[0]], dvd[b],
                              gsem[b]).wait()
        for i in range(8):
            sl = pl.ds(16 * i, 16)
            nrm[b][sl] = dvs[b][sl] * ew[b][0, sl] * dvd[b][sl]

        def grp(q, _):
            n16 = nrm[b][pl.ds(16 * q, 16)]
            for e0 in range(16):
                nb = _bcast(n16, e0)
                r = 16 * q + e0
                for j in range(8):
                    sl2 = pl.ds(16 * j, 16)
                    rows[b][r, sl2] = rows[b][r, sl2] * nb
            return 0

        lax.fori_loop(0, 8, grp, 0)
        pltpu.async_copy(rows[b], acc_s.at[di[b].at[0]], ssem[b], add=True)

    fire(0, 0)

    def pair(t, _):
        k0 = 2 * t
        k1 = k0 + 1
        pl.when(k1 < trips)(lambda: fire(k1, 1))
        consume(0)
        pl.when(k1 < trips)(lambda: consume(1))
        pl.when(k0 + 2 < trips)(lambda: fire(k0 + 2, 0))
        return 0

    lax.fori_loop(0, (trips + 1) // 2, pair, 0)

    pltpu.make_async_copy(rows0, acc_s.at[di0.at[0]], ssem0).wait()
    pl.when(trips >= 2)(lambda: pltpu.make_async_copy(
        rows1, acc_s.at[di1.at[0]], ssem1).wait())
    plsc.subcore_barrier()

    pl.when(sid < 15)(lambda: pltpu.sync_copy(
        acc_s.at[pl.ds(640 * sid, 640)], parts.at[cid, pl.ds(640 * sid, 640)]))
    pl.when(sid == 15)(lambda: pltpu.sync_copy(
        acc_s.at[pl.ds(9600, 400)], parts.at[cid, pl.ds(9600, 400)]))


def _sc_conv(src_all, dst_all, ew_all, h_all):
    mesh = plsc.VectorSubcoreMesh(core_axis_name="c", subcore_axis_name="s")
    f32 = jnp.float32
    i32 = jnp.int32
    idx_sets = []
    for _ in range(2):
        idx_sets += [
            pltpu.VMEM((1, 128), i32),
            pltpu.VMEM((1, 128), i32),
            pltpu.VMEM((1, 128), f32),
            pltpu.VMEM((1, 128), i32),
            pltpu.VMEM((1, 128), i32),
            pltpu.VMEM((1, 128), i32),
            pltpu.VMEM((128,), f32),
            pltpu.VMEM((128,), f32),
            pltpu.VMEM((128,), f32),
            pltpu.VMEM((K, D), f32),
        ]
    return pl.kernel(
        _sc_body,
        out_type=[jax.ShapeDtypeStruct((NC, N, D), f32),
                  jax.ShapeDtypeStruct((2 * NC * NPAD,), f32)],
        mesh=mesh,
        compiler_params=pltpu.CompilerParams(use_tc_tiling_on_sc=False,
                                             needs_layout_passes=False),
        scratch_types=[
            pltpu.VMEM_SHARED((NPAD, D), f32),
            pltpu.VMEM_SHARED((NPAD,), f32),
            pltpu.VMEM_SHARED((NPAD,), f32),
            pltpu.VMEM((640,), f32),
            pltpu.VMEM((1280,), f32),
            pltpu.VMEM((128,), f32),
            pltpu.VMEM((1, 128), i32),
            pltpu.VMEM((2, 128), i32),
            pltpu.VMEM((2, 128), i32),
            pltpu.VMEM((2, 128), f32),
            pltpu.VMEM((2, 128), f32),
        ] + idx_sets + [
            pltpu.SemaphoreType.DMA,
            pltpu.SemaphoreType.DMA,
            pltpu.SemaphoreType.DMA,
            pltpu.SemaphoreType.DMA,
            pltpu.SemaphoreType.DMA,
            pltpu.SemaphoreType.DMA,
            pltpu.SemaphoreType.DMA,
        ],
    )(src_all, dst_all, ew_all, h_all)


def _mm_body(a_ref, l_ref, w_ref, h_ref):
    xa = a_ref[...] + l_ref[...]
    h_ref[0] = jnp.dot(xa, w_ref[0], preferred_element_type=jnp.float32)


def _tc_matmul(activity, learning, Ws):
    BM = 1000
    return pl.pallas_call(
        _mm_body,
        grid=(NC, N // BM),
        in_specs=[
            pl.BlockSpec((BM, D), lambda g, i: (i, 0)),
            pl.BlockSpec((BM, D), lambda g, i: (i, 0)),
            pl.BlockSpec((1, D, D), lambda g, i: (g, 0, 0)),
        ],
        out_specs=pl.BlockSpec((1, BM, D), lambda g, i: (g, i, 0)),
        out_shape=jax.ShapeDtypeStruct((NC, N, D), jnp.float32),
    )(activity, learning, Ws)


def _fin_body(p0_ref, p1_ref, a_ref, l_ref, bl_ref, bg_ref, o_ref):
    xlg = p0_ref[...] + p1_ref[...] + (bl_ref[...] + bg_ref[...])
    wei = jax.nn.sigmoid(xlg)
    o_ref[...] = 2.0 * a_ref[...] * wei + 2.0 * l_ref[...] * (1.0 - wei)


def _tc_final(p0, p1, activity, learning, b_l2, b_g2):
    BM = 1000
    row = lambda i: (i, 0)
    fix = lambda i: (0, 0)
    return pl.pallas_call(
        _fin_body,
        grid=(N // BM,),
        in_specs=[
            pl.BlockSpec((BM, D), row),
            pl.BlockSpec((BM, D), row),
            pl.BlockSpec((BM, D), row),
            pl.BlockSpec((BM, D), row),
            pl.BlockSpec((1, D), fix),
            pl.BlockSpec((1, D), fix),
        ],
        out_specs=pl.BlockSpec((BM, D), row),
        out_shape=jax.ShapeDtypeStruct((N, D), jnp.float32),
    )(p0, p1, activity, learning, b_l2, b_g2)


def kernel(activity, learning, edge_index, edge_weight,
           global_edge_index, global_edge_weight,
           W_l1, b_l1, W_l2, b_l2, W_g1, b_g1, W_g2, b_g2):
    h = _tc_matmul(activity, learning, jnp.stack([W_l2, W_g2]))
    h_all = h.reshape(NC * N, D)

    src_all = jnp.concatenate(
        [edge_index[0], global_edge_index[0]]).reshape(NCH, 128)
    dst_all = jnp.concatenate(
        [edge_index[1], global_edge_index[1]]).reshape(NCH, 128)
    ew_all = jnp.concatenate(
        [edge_weight, global_edge_weight]).reshape(NCH, 128)

    parts, _ = _sc_conv(src_all, dst_all, ew_all, h_all)

    return _tc_final(parts[0], parts[1], activity, learning,
                     b_l2.reshape(1, D), b_g2.reshape(1, D))


# speedup vs baseline: 25.1122x; 1.0006x over previous
import jax
import jax.numpy as jnp
from jax import lax
from jax.experimental import pallas as pl
from jax.experimental.pallas import tpu as pltpu
from jax.experimental.pallas import tpu_sc as plsc

N = 10000
D = 128
EL = 320000
EG = 160000
NC = 2
NS = 16
NPAD = 10240
K = 128
NCH_L = EL // K
NCH_G = EG // K
NCH = NCH_L + NCH_G
CPS = NCH // NC
DG = NCH // 2


def _rsqrt16(x):
    i = plsc.bitcast(x, jnp.int32)
    y = plsc.bitcast(jnp.int32(0x5F3759DF) - lax.shift_right_logical(i, 1),
                     jnp.float32)
    for _ in range(3):
        y = y * (1.5 - 0.5 * x * y * y)
    return y


def _bcast(v16, lane):
    return jnp.take_along_axis(v16, jnp.full((16,), lane, jnp.int32), axis=0,
                               mode="promise_in_bounds")


def _sc_body(src_all, dst_all, ew_all, h_all, parts, dinv_hbm, *s):
    (acc_s, dgl_s, dgg_s, zv, nbuf, dvrow, ii,
     dd0, dd1, ee0, ee1,
     si0, di0, ew0, hix0, six0, dix0, dvs0, dvd0, nrm0, rows0,
     si1, di1, ew1, hix1, six1, dix1, dvs1, dvd1, nrm1, rows1,
     isem0, isem1, gsem0, gsem1, ssem0, ssem1, zsem) = s
    dd = (dd0, dd1)
    ee = (ee0, ee1)
    si = (si0, si1)
    di = (di0, di1)
    ew = (ew0, ew1)
    hix = (hix0, hix1)
    six = (six0, six1)
    dix = (dix0, dix1)
    dvs = (dvs0, dvs1)
    dvd = (dvd0, dvd1)
    nrm = (nrm0, nrm1)
    rows = (rows0, rows1)
    isem = (isem0, isem1)
    gsem = (gsem0, gsem1)
    ssem = (ssem0, ssem1)

    cid = lax.axis_index("c")
    sid = lax.axis_index("s")

    z16 = jnp.zeros((16,), jnp.float32)
    for i in range(40):
        zv[pl.ds(16 * i, 16)] = z16
    pltpu.sync_copy(zv, dgl_s.at[pl.ds(640 * sid, 640)])
    pltpu.sync_copy(zv, dgg_s.at[pl.ds(640 * sid, 640)])

    def zrow(r, _):
        for j in range(8):
            rows0[r, pl.ds(16 * j, 16)] = z16
        return 0

    lax.fori_loop(0, 128, zrow, 0)
    zcps = [pltpu.async_copy(rows0, acc_s.at[pl.ds(640 * sid + 128 * t, 128)],
                             zsem) for t in range(5)]
    plsc.subcore_barrier()

    scope_deg = jax.named_scope("deg_phase")
    scope_deg.__enter__()
    dtrips = (DG - sid + NS - 1) // NS

    def dfire(k, b):
        g = sid + NS * k
        pltpu.async_copy(dst_all.at[pl.ds(2 * g, 2)], dd[b], isem[b])
        pltpu.async_copy(ew_all.at[pl.ds(2 * g, 2)], ee[b], isem[b])

    def dconsume(k, b):
        g = sid + NS * k
        pltpu.make_async_copy(dst_all.at[pl.ds(2 * g, 2)], dd[b],
                              isem[b]).wait()
        pltpu.make_async_copy(ew_all.at[pl.ds(2 * g, 2)], ee[b],
                              isem[b]).wait()

        def to(tgt):
            for j in range(2):
                pltpu.sync_copy(ee[b].at[j], tgt.at[dd[b].at[j]], add=True)

        pl.when(g < NCH_L // 2)(lambda: to(dgl_s))
        pl.when(g >= NCH_L // 2)(lambda: to(dgg_s))

    dfire(0, 0)

    def dpair(t, _):
        k0 = 2 * t
        k1 = k0 + 1
        pl.when(k1 < dtrips)(lambda: dfire(k1, 1))
        dconsume(k0, 0)
        pl.when(k0 + 2 < dtrips)(lambda: dfire(k0 + 2, 0))
        pl.when(k1 < dtrips)(lambda: dconsume(k1, 1))
        return 0

    lax.fori_loop(0, (dtrips + 1) // 2, dpair, 0)

    for cp in zcps:
        cp.wait()
    plsc.subcore_barrier()
    scope_deg.__exit__(None, None, None)

    pl.when(sid < 8)(
        lambda: pltpu.sync_copy(dgl_s.at[pl.ds(1280 * sid, 1280)], nbuf))
    pl.when(sid >= 8)(
        lambda: pltpu.sync_copy(dgg_s.at[pl.ds(1280 * sid - NPAD, 1280)],
                                nbuf))

    def nt(i, _):
        sl = pl.ds(16 * i, 16)
        nbuf[sl] = _rsqrt16(nbuf[sl] + 1.0)
        return 0

    lax.fori_loop(0, 80, nt, 0)
    pltpu.sync_copy(nbuf,
                    dinv_hbm.at[pl.ds(cid * 2 * NPAD + 1280 * sid, 1280)])
    plsc.subcore_barrier()

    scope_sl = jax.named_scope("selfloop")
    scope_sl.__enter__()
    offh = cid * N
    offd = cid * (2 * NPAD) + cid * NPAD
    iota16 = lax.broadcasted_iota(jnp.int32, (16,), 0)
    for t in range(5):
        base = 640 * sid + 128 * t
        pltpu.sync_copy(dinv_hbm.at[pl.ds(offd + base, 128)], dvrow)
        for i in range(8):
            sl = pl.ds(16 * i, 16)
            row16 = iota16 + (base + 16 * i)
            ii[0, sl] = row16
            hix0[0, sl] = jnp.minimum(row16, N - 1) + offh
        pltpu.async_copy(h_all.at[hix0.at[0]], rows0, gsem0).wait()

        def srow2(q, _):
            d16 = dvrow[pl.ds(16 * q, 16)]
            sq = d16 * d16
            for e0 in range(16):
                nb = _bcast(sq, e0)
                r = 16 * q + e0
                for j in range(8):
                    sl2 = pl.ds(16 * j, 16)
                    rows0[r, sl2] = rows0[r, sl2] * nb
            return 0

        lax.fori_loop(0, 8, srow2, 0)
        pltpu.sync_copy(rows0, acc_s.at[ii.at[0]], add=True)

    scope_sl.__exit__(None, None, None)
    scope_msg = jax.named_scope("msg_phase")
    scope_msg.__enter__()
    cbase = cid * CPS + sid
    trips = (CPS - sid + NS - 1) // NS

    def fire(k, b):
        pl.when(k >= 2)(lambda: pltpu.make_async_copy(
            rows[b], acc_s.at[di[b].at[0]], ssem[b]).wait())
        c = cbase + NS * k
        a0 = pltpu.async_copy(src_all.at[pl.ds(c, 1)], si[b], isem[b])
        a1 = pltpu.async_copy(dst_all.at[pl.ds(c, 1)], di[b], isem[b])
        a2 = pltpu.async_copy(ew_all.at[pl.ds(c, 1)], ew[b], isem[b])
        a0.wait()
        a1.wait()
        a2.wait()
        g = jnp.where(c >= NCH_L, jnp.int32(1), jnp.int32(0))
        oh = g * N
        od = g * NPAD + cid * (2 * NPAD)
        for i in range(8):
            sl = pl.ds(16 * i, 16)
            s16 = si[b][0, sl]
            d16 = di[b][0, sl]
            hix[b][0, sl] = s16 + oh
            six[b][0, sl] = s16 + od
            dix[b][0, sl] = d16 + od
        pltpu.async_copy(h_all.at[hix[b].at[0]], rows[b], gsem[b])
        pltpu.async_copy(dinv_hbm.at[six[b].at[0]], dvs[b], gsem[b])
        pltpu.async_copy(dinv_hbm.at[dix[b].at[0]], dvd[b], gsem[b])

    def consume(b):
        pltpu.make_async_copy(h_all.at[hix[b].at[0]], rows[b],
                              gsem[b]).wait()
        pltpu.make_async_copy(dinv_hbm.at[six[b].at[0]], dvs[b],
                              gsem[b]).wait()
        pltpu.make_async_copy(dinv_hbm.at[dix[b].at[0]], dvd[b],
                              gsem[b]).wait()
        for i in range(8):
            sl = pl.ds(16 * i, 16)
            nrm[b][sl] = dvs[b][sl] * ew[b][0, sl] * dvd[b][sl]

        def grp(q, _):
            n16 = nrm[b][pl.ds(16 * q, 16)]
            for e0 in range(16):
                nb = _bcast(n16, e0)
                r = 16 * q + e0
                for j in range(8):
                    sl2 = pl.ds(16 * j, 16)
                    rows[b][r, sl2] = rows[b][r, sl2] * nb
            return 0

        lax.fori_loop(0, 8, grp, 0)
        pltpu.async_copy(rows[b], acc_s.at[di[b].at[0]], ssem[b], add=True)

    fire(0, 0)

    def pair(t, _):
        k0 = 2 * t
        k1 = k0 + 1
        pl.when(k1 < trips)(lambda: fire(k1, 1))
        consume(0)
        pl.when(k1 < trips)(lambda: consume(1))
        pl.when(k0 + 2 < trips)(lambda: fire(k0 + 2, 0))
        return 0

    lax.fori_loop(0, (trips + 1) // 2, pair, 0)

    pltpu.make_async_copy(rows0, acc_s.at[di0.at[0]], ssem0).wait()
    pl.when(trips >= 2)(lambda: pltpu.make_async_copy(
        rows1, acc_s.at[di1.at[0]], ssem1).wait())
    plsc.subcore_barrier()
    scope_msg.__exit__(None, None, None)

    pl.when(sid < 15)(lambda: pltpu.sync_copy(
        acc_s.at[pl.ds(640 * sid, 640)], parts.at[cid, pl.ds(640 * sid, 640)]))
    pl.when(sid == 15)(lambda: pltpu.sync_copy(
        acc_s.at[pl.ds(9600, 400)], parts.at[cid, pl.ds(9600, 400)]))


def _sc_conv(src_all, dst_all, ew_all, h_all):
    mesh = plsc.VectorSubcoreMesh(core_axis_name="c", subcore_axis_name="s")
    f32 = jnp.float32
    i32 = jnp.int32
    idx_sets = []
    for _ in range(2):
        idx_sets += [
            pltpu.VMEM((1, 128), i32),
            pltpu.VMEM((1, 128), i32),
            pltpu.VMEM((1, 128), f32),
            pltpu.VMEM((1, 128), i32),
            pltpu.VMEM((1, 128), i32),
            pltpu.VMEM((1, 128), i32),
            pltpu.VMEM((128,), f32),
            pltpu.VMEM((128,), f32),
            pltpu.VMEM((128,), f32),
            pltpu.VMEM((K, D), f32),
        ]
    return pl.kernel(
        _sc_body,
        out_type=[jax.ShapeDtypeStruct((NC, N, D), f32),
                  jax.ShapeDtypeStruct((2 * NC * NPAD,), f32)],
        mesh=mesh,
        compiler_params=pltpu.CompilerParams(use_tc_tiling_on_sc=False,
                                             needs_layout_passes=False),
        scratch_types=[
            pltpu.VMEM_SHARED((NPAD, D), f32),
            pltpu.VMEM_SHARED((NPAD,), f32),
            pltpu.VMEM_SHARED((NPAD,), f32),
            pltpu.VMEM((640,), f32),
            pltpu.VMEM((1280,), f32),
            pltpu.VMEM((128,), f32),
            pltpu.VMEM((1, 128), i32),
            pltpu.VMEM((2, 128), i32),
            pltpu.VMEM((2, 128), i32),
            pltpu.VMEM((2, 128), f32),
            pltpu.VMEM((2, 128), f32),
        ] + idx_sets + [
            pltpu.SemaphoreType.DMA,
            pltpu.SemaphoreType.DMA,
            pltpu.SemaphoreType.DMA,
            pltpu.SemaphoreType.DMA,
            pltpu.SemaphoreType.DMA,
            pltpu.SemaphoreType.DMA,
            pltpu.SemaphoreType.DMA,
        ],
    )(src_all, dst_all, ew_all, h_all)


def _mm_body(a_ref, l_ref, w_ref, h_ref):
    xa = a_ref[...] + l_ref[...]
    h_ref[0] = jnp.dot(xa, w_ref[0], preferred_element_type=jnp.float32)


def _tc_matmul(activity, learning, Ws):
    BM = 1000
    return pl.pallas_call(
        _mm_body,
        grid=(NC, N // BM),
        in_specs=[
            pl.BlockSpec((BM, D), lambda g, i: (i, 0)),
            pl.BlockSpec((BM, D), lambda g, i: (i, 0)),
            pl.BlockSpec((1, D, D), lambda g, i: (g, 0, 0)),
        ],
        out_specs=pl.BlockSpec((1, BM, D), lambda g, i: (g, i, 0)),
        out_shape=jax.ShapeDtypeStruct((NC, N, D), jnp.float32),
    )(activity, learning, Ws)


def _fin_body(p0_ref, p1_ref, a_ref, l_ref, bl_ref, bg_ref, o_ref):
    xlg = p0_ref[...] + p1_ref[...] + (bl_ref[...] + bg_ref[...])
    wei = jax.nn.sigmoid(xlg)
    o_ref[...] = 2.0 * a_ref[...] * wei + 2.0 * l_ref[...] * (1.0 - wei)


def _tc_final(p0, p1, activity, learning, b_l2, b_g2):
    BM = 1000
    row = lambda i: (i, 0)
    fix = lambda i: (0, 0)
    return pl.pallas_call(
        _fin_body,
        grid=(N // BM,),
        in_specs=[
            pl.BlockSpec((BM, D), row),
            pl.BlockSpec((BM, D), row),
            pl.BlockSpec((BM, D), row),
            pl.BlockSpec((BM, D), row),
            pl.BlockSpec((1, D), fix),
            pl.BlockSpec((1, D), fix),
        ],
        out_specs=pl.BlockSpec((BM, D), row),
        out_shape=jax.ShapeDtypeStruct((N, D), jnp.float32),
    )(p0, p1, activity, learning, b_l2, b_g2)


def kernel(activity, learning, edge_index, edge_weight,
           global_edge_index, global_edge_weight,
           W_l1, b_l1, W_l2, b_l2, W_g1, b_g1, W_g2, b_g2):
    h = _tc_matmul(activity, learning, jnp.stack([W_l2, W_g2]))
    h_all = h.reshape(NC * N, D)

    src_all = jnp.concatenate(
        [edge_index[0], global_edge_index[0]]).reshape(NCH, 128)
    dst_all = jnp.concatenate(
        [edge_index[1], global_edge_index[1]]).reshape(NCH, 128)
    ew_all = jnp.concatenate(
        [edge_weight, global_edge_weight]).reshape(NCH, 128)

    parts, _ = _sc_conv(src_all, dst_all, ew_all, h_all)

    return _tc_final(parts[0], parts[1], activity, learning,
                     b_l2.reshape(1, D), b_g2.reshape(1, D))
